# Initial kernel scaffold; baseline (speedup 1.0000x reference)
import jax
import jax.numpy as jnp
from jax.experimental import pallas as pl


def kernel(x, edge_index, edge_attr, batch_q_embds, W_pr1, b_pr1, W_pr2, b_pr2, W_s1, b_s1, W_n1, W_s2, b_s2, W_n2, W_p1, b_p1, W_p2, b_p2):
    raise NotImplementedError("write your pallas kernel here")


# speedup vs baseline: 5.2242x; 5.2242x over previous
import functools

import jax
import jax.numpy as jnp
from jax import lax
from jax.experimental import pallas as pl
from jax.experimental.pallas import tpu as pltpu
from jax.experimental.pallas import tpu_sc as plsc

_N = 10000
_E = 320000
_D = 128
_K = 1024
_NP = 10240
_CHUNK = 128
_NCH = _E // _CHUNK
_NW = 32
_RPS = _NP // 16

def _worker_id():
  ci = lax.axis_index("c")
  si = lax.axis_index("s")
  return si * 2 + ci, ci, si


def _chunk_count(w):
  return 78 + jnp.where(w < 4, 1, 0).astype(jnp.int32)


def _sc_edge_agg_body(ea_hbm, earev_hbm, hid_hbm, tid_hbm, zr_hbm, zd_hbm,
                      acc_out, deg_out,
                      idx_v, rows_v, ones_v, dbuf_v, acc_sh, deg_sh):
  w, ci, si = _worker_id()
  for j in range(8):
    ones_v[pl.ds(j * 16, 16)] = jnp.ones((16,), jnp.float32)
  pltpu.sync_copy(zr_hbm, rows_v)
  for k in range(_RPS // _CHUNK):
    pltpu.sync_copy(rows_v, acc_sh.at[pl.ds(si * _RPS + k * _CHUNK, _CHUNK), :])
  pltpu.sync_copy(zd_hbm, dbuf_v)
  pltpu.sync_copy(dbuf_v, deg_sh.at[pl.ds(si * _RPS, _RPS)])
  plsc.subcore_barrier()

  def body(j, carry):
    base = (w + _NW * j) * _CHUNK
    pltpu.sync_copy(tid_hbm.at[pl.ds(base, _CHUNK)], idx_v)
    pltpu.sync_copy(ea_hbm.at[pl.ds(base, _CHUNK), :], rows_v)
    pltpu.sync_copy(rows_v, acc_sh.at[idx_v], add=True)
    pltpu.sync_copy(ones_v, deg_sh.at[idx_v], add=True)
    pltpu.sync_copy(hid_hbm.at[pl.ds(base, _CHUNK)], idx_v)
    pltpu.sync_copy(earev_hbm.at[pl.ds(base, _CHUNK), :], rows_v)
    pltpu.sync_copy(rows_v, acc_sh.at[idx_v], add=True)
    pltpu.sync_copy(ones_v, deg_sh.at[idx_v], add=True)
    return carry

  lax.fori_loop(0, _chunk_count(w), body, 0)
  plsc.subcore_barrier()
  for k in range(_RPS // _CHUNK):
    r0 = si * _RPS + k * _CHUNK
    pltpu.sync_copy(acc_sh.at[pl.ds(r0, _CHUNK), :], rows_v)
    pltpu.sync_copy(rows_v, acc_out.at[ci, pl.ds(r0, _CHUNK), :])
  pltpu.sync_copy(deg_sh.at[pl.ds(si * _RPS, _RPS)], dbuf_v)
  pltpu.sync_copy(dbuf_v, deg_out.at[ci, pl.ds(si * _RPS, _RPS)])


def _sc_spmm_body(h_hbm, hid_hbm, tid_hbm, zr_hbm,
                  acc_out,
                  idxh_v, idxt_v, bufh_v, buft_v, acc_sh, sem1, sem2):
  w, ci, si = _worker_id()
  pltpu.sync_copy(zr_hbm, bufh_v)
  for k in range(_RPS // _CHUNK):
    pltpu.sync_copy(bufh_v, acc_sh.at[pl.ds(si * _RPS + k * _CHUNK, _CHUNK), :])
  plsc.subcore_barrier()

  def body(j, carry):
    base = (w + _NW * j) * _CHUNK
    pltpu.sync_copy(hid_hbm.at[pl.ds(base, _CHUNK)], idxh_v)
    pltpu.sync_copy(tid_hbm.at[pl.ds(base, _CHUNK)], idxt_v)
    cp1 = pltpu.async_copy(h_hbm.at[idxh_v], bufh_v, sem1)
    cp2 = pltpu.async_copy(h_hbm.at[idxt_v], buft_v, sem2)
    cp1.wait()
    cp2.wait()
    pltpu.sync_copy(bufh_v, acc_sh.at[idxt_v], add=True)
    pltpu.sync_copy(buft_v, acc_sh.at[idxh_v], add=True)
    return carry

  lax.fori_loop(0, _chunk_count(w), body, 0)
  plsc.subcore_barrier()
  for k in range(_RPS // _CHUNK):
    r0 = si * _RPS + k * _CHUNK
    pltpu.sync_copy(acc_sh.at[pl.ds(r0, _CHUNK), :], bufh_v)
    pltpu.sync_copy(bufh_v, acc_out.at[ci, pl.ds(r0, _CHUNK), :])


def _sc_gather2_body(ph_hbm, pt_hbm, hid_hbm, tid_hbm,
                     g_out,
                     idxh_v, idxt_v, buf_v, sem):
  w, ci, si = _worker_id()

  def body(j, carry):
    base = (w + _NW * j) * _CHUNK
    pltpu.sync_copy(hid_hbm.at[pl.ds(base, _CHUNK)], idxh_v)
    pltpu.sync_copy(tid_hbm.at[pl.ds(base, _CHUNK)], idxt_v)
    pltpu.async_copy(ph_hbm.at[idxh_v], buf_v, sem).wait()
    pltpu.async_copy(pt_hbm.at[idxt_v], buf_v, sem, add=True).wait()
    pltpu.sync_copy(buf_v, g_out.at[pl.ds(base, _CHUNK), :])
    return carry

  lax.fori_loop(0, _chunk_count(w), body, 0)


@functools.cache
def _sc_kernels():
  mesh = plsc.VectorSubcoreMesh(core_axis_name="c", subcore_axis_name="s",
                                num_cores=2, num_subcores=16)
  edge_agg = functools.partial(
      pl.kernel,
      mesh=mesh,
      out_type=(
          jax.ShapeDtypeStruct((2, _NP, _D), jnp.float32),
          jax.ShapeDtypeStruct((2, _NP), jnp.float32),
      ),
      scratch_types=[
          pltpu.VMEM((_CHUNK,), jnp.int32),
          pltpu.VMEM((_CHUNK, _D), jnp.float32),
          pltpu.VMEM((_CHUNK,), jnp.float32),
          pltpu.VMEM((_RPS,), jnp.float32),
          pltpu.VMEM_SHARED((_NP, _D), jnp.float32),
          pltpu.VMEM_SHARED((_NP,), jnp.float32),
      ],
  )(_sc_edge_agg_body)
  spmm = functools.partial(
      pl.kernel,
      mesh=mesh,
      out_type=jax.ShapeDtypeStruct((2, _NP, _D), jnp.float32),
      scratch_types=[
          pltpu.VMEM((_CHUNK,), jnp.int32),
          pltpu.VMEM((_CHUNK,), jnp.int32),
          pltpu.VMEM((_CHUNK, _D), jnp.float32),
          pltpu.VMEM((_CHUNK, _D), jnp.float32),
          pltpu.VMEM_SHARED((_NP, _D), jnp.float32),
          pltpu.SemaphoreType.DMA,
          pltpu.SemaphoreType.DMA,
      ],
  )(_sc_spmm_body)
  gather2 = functools.partial(
      pl.kernel,
      mesh=mesh,
      out_type=jax.ShapeDtypeStruct((_E, _D), jnp.float32),
      scratch_types=[
          pltpu.VMEM((_CHUNK,), jnp.int32),
          pltpu.VMEM((_CHUNK,), jnp.int32),
          pltpu.VMEM((_CHUNK, _D), jnp.float32),
          pltpu.SemaphoreType.DMA,
      ],
  )(_sc_gather2_body)
  return edge_agg, spmm, gather2


def _mlp2_body(x_ref, w1_ref, b1_ref, w2_ref, b2_ref, o_ref):
  h = jnp.maximum(
      jnp.dot(x_ref[...], w1_ref[...], preferred_element_type=jnp.float32, precision=lax.Precision.DEFAULT)
      + b1_ref[...], 0.0)
  o_ref[...] = (
      jnp.dot(h, w2_ref[...], preferred_element_type=jnp.float32, precision=lax.Precision.DEFAULT)
      + b2_ref[...])


def _edge_mlp(ea, w1, b1, w2, b2, rb):
  n = ea.shape[0]
  full = lambda i: (0, 0)
  return pl.pallas_call(
      _mlp2_body,
      grid=(n // rb,),
      in_specs=[
          pl.BlockSpec((rb, _D), lambda i: (i, 0)),
          pl.BlockSpec((_D, _D), full),
          pl.BlockSpec((1, _D), full),
          pl.BlockSpec((_D, _D), full),
          pl.BlockSpec((1, _D), full),
      ],
      out_specs=pl.BlockSpec((rb, _D), lambda i: (i, 0)),
      out_shape=jax.ShapeDtypeStruct((n, _D), jnp.float32),
  )(ea, w1, b1, w2, b2)


def _layer_body(h_ref, a0, a1, e0, e1, d0, d1, ws, wn, b, o_ref):
  deg = jnp.maximum(d0[...] + d1[...], 1.0)
  agg = (a0[...] + a1[...] + e0[...] + e1[...]) / deg
  o_ref[...] = jnp.maximum(
      jnp.dot(h_ref[...], ws[...], preferred_element_type=jnp.float32, precision=lax.Precision.DEFAULT)
      + jnp.dot(agg, wn[...], preferred_element_type=jnp.float32, precision=lax.Precision.DEFAULT)
      + b[...], 0.0)


def _layer_combine(h, a0, a1, e0, e1, d0, d1, ws, wn, b, rb):
  full = lambda i: (0, 0)
  blk = pl.BlockSpec((rb, _D), lambda i: (i, 0))
  col = pl.BlockSpec((rb, 1), lambda i: (i, 0))
  return pl.pallas_call(
      _layer_body,
      grid=(_NP // rb,),
      in_specs=[blk, blk, blk, blk, blk, col, col,
                pl.BlockSpec((_D, _D), full),
                pl.BlockSpec((_D, _D), full),
                pl.BlockSpec((1, _D), full)],
      out_specs=blk,
      out_shape=jax.ShapeDtypeStruct((_NP, _D), jnp.float32),
  )(h, a0, a1, e0, e1, d0, d1, ws, wn, b)


def _layer2_body(h1_ref, a0, a1, e0, e1, d0, d1, ws, wn, b,
                 wa, wb, wc, wd, ph_ref, pt_ref):
  deg = jnp.maximum(d0[...] + d1[...], 1.0)
  agg = (a0[...] + a1[...] + e0[...] + e1[...]) / deg
  h1 = h1_ref[...]
  h2 = jnp.maximum(
      jnp.dot(h1, ws[...], preferred_element_type=jnp.float32, precision=lax.Precision.DEFAULT)
      + jnp.dot(agg, wn[...], preferred_element_type=jnp.float32, precision=lax.Precision.DEFAULT)
      + b[...], 0.0)
  ph_ref[...] = (
      jnp.dot(h1, wa[...], preferred_element_type=jnp.float32, precision=lax.Precision.DEFAULT)
      + jnp.dot(h2, wb[...], preferred_element_type=jnp.float32, precision=lax.Precision.DEFAULT))
  pt_ref[...] = (
      jnp.dot(h1, wc[...], preferred_element_type=jnp.float32, precision=lax.Precision.DEFAULT)
      + jnp.dot(h2, wd[...], preferred_element_type=jnp.float32, precision=lax.Precision.DEFAULT))


def _layer2_combine(h1, a0, a1, e0, e1, d0, d1, ws, wn, b, wa, wb, wc, wd, rb):
  full = lambda i: (0, 0)
  blk = pl.BlockSpec((rb, _D), lambda i: (i, 0))
  col = pl.BlockSpec((rb, 1), lambda i: (i, 0))
  wspec = pl.BlockSpec((_D, _D), full)
  out = jax.ShapeDtypeStruct((_NP, _D), jnp.float32)
  return pl.pallas_call(
      _layer2_body,
      grid=(_NP // rb,),
      in_specs=[blk, blk, blk, blk, blk, col, col,
                wspec, wspec, pl.BlockSpec((1, _D), full),
                wspec, wspec, wspec, wspec],
      out_specs=(blk, blk),
      out_shape=(out, out),
  )(h1, a0, a1, e0, e1, d0, d1, ws, wn, b, wa, wb, wc, wd)


def _score_body(bq_ref, ea_ref, g_ref, gn_ref, wq, we, b1, w2, b2,
                logit_ref, y_ref):
  hid = jnp.maximum(
      jnp.dot(bq_ref[...], wq[...], preferred_element_type=jnp.float32, precision=lax.Precision.DEFAULT)
      + jnp.dot(ea_ref[...], we[...], preferred_element_type=jnp.float32, precision=lax.Precision.DEFAULT)
      + g_ref[...] + b1[...], 0.0)
  lg = jnp.dot(hid, w2[...], preferred_element_type=jnp.float32, precision=lax.Precision.DEFAULT) + b2[...]
  logit_ref[...] = lg
  y_ref[...] = lg + gn_ref[...]


def _edge_score(bq, ea, g, gn, wq, we, b1, w2, b2, rb):
  full = lambda i: (0, 0)
  blk = pl.BlockSpec((rb, _D), lambda i: (i, 0))
  col = pl.BlockSpec((rb, 1), lambda i: (i, 0))
  out = jax.ShapeDtypeStruct((_E, 1), jnp.float32)
  return pl.pallas_call(
      _score_body,
      grid=(_E // rb,),
      in_specs=[blk, blk, blk, col,
                pl.BlockSpec((_D, _D), full),
                pl.BlockSpec((_D, _D), full),
                pl.BlockSpec((1, _D), full),
                pl.BlockSpec((_D, 1), full),
                pl.BlockSpec((1, 1), full)],
      out_specs=(col, col),
      out_shape=(out, out),
  )(bq, ea, g, gn, wq, we, b1, w2, b2)


_EPAD = 2560 * 128


def _topk_body(y_ref, o_ref):
  zi = lax.bitcast_convert_type(y_ref[...], jnp.int32)
  z = zi ^ ((zi >> 31) & jnp.int32(0x7FFFFFFF))

  def bis1(i, c):
    lo, hi = c
    half = (lo >> 1) + (hi >> 1)
    mid0 = half + (lo & hi & 1) + ((lo ^ hi) & 1)
    mid = jnp.where(lo < hi, mid0, lo)
    cnt = jnp.sum((z >= mid).astype(jnp.int32))
    pred = cnt >= _K
    return (jnp.where(pred, mid, lo), jnp.where(pred, hi, mid - 1))

  lo, _ = lax.fori_loop(0, 33, bis1, (jnp.int32(-(2 ** 31)),
                                      jnp.int32(2 ** 31 - 1)))
  t = lo
  c_gt = jnp.sum((z > t).astype(jnp.int32))
  r = _K - c_gt
  eq = z == t
  rows = lax.broadcasted_iota(jnp.int32, z.shape, 0)
  cols = lax.broadcasted_iota(jnp.int32, z.shape, 1)
  flat = rows * 128 + cols

  def bis2(i, c):
    lo, hi = c
    mid = jnp.where(lo < hi, (lo + hi) >> 1, lo)
    cnt = jnp.sum((eq & (flat <= mid)).astype(jnp.int32))
    ok = cnt >= r
    return (jnp.where(ok, lo, mid + 1), jnp.where(ok, mid, hi))

  m, _ = lax.fori_loop(0, 20, bis2, (jnp.int32(0), jnp.int32(_EPAD - 1)))
  o_ref[...] = ((z > t) | (eq & (flat <= m))).astype(jnp.float32)


def _topk_mask(ypad):
  return pl.pallas_call(
      _topk_body,
      out_shape=jax.ShapeDtypeStruct((_EPAD // 128, 128), jnp.float32),
  )(ypad)


def kernel(x, edge_index, edge_attr, batch_q_embds, W_pr1, b_pr1, W_pr2,
           b_pr2, W_s1, b_s1, W_n1, W_s2, b_s2, W_n2, W_p1, b_p1, W_p2, b_p2):
  h_id = edge_index[0]
  t_id = edge_index[1]
  r1 = lambda v: v.reshape(1, -1)

  u = jax.random.uniform(jax.random.key(42), (_E,), jnp.float32,
                         1e-10, 1.0 - 1e-10)
  gn = (-jnp.log(-jnp.log(u))).reshape(_E, 1)

  _sc_edge_agg, _sc_spmm, _sc_gather2 = _sc_kernels()

  ea_rev = _edge_mlp(edge_attr, W_pr1, r1(b_pr1), W_pr2, r1(b_pr2), 2560)

  zr = jnp.zeros((_CHUNK, _D), jnp.float32)
  zd = jnp.zeros((_RPS,), jnp.float32)
  eagg, deg = _sc_edge_agg(edge_attr, ea_rev, h_id, t_id, zr, zd)
  d0 = deg[0].reshape(_NP, 1)
  d1 = deg[1].reshape(_NP, 1)

  xp = jnp.zeros((_NP, _D), jnp.float32).at[:_N].set(x)
  acc1 = _sc_spmm(xp, h_id, t_id, zr)
  h1 = _layer_combine(xp, acc1[0], acc1[1], eagg[0], eagg[1], d0, d1,
                      W_s1, W_n1, r1(b_s1), 1280)

  acc2 = _sc_spmm(h1, h_id, t_id, zr)
  ph, pt = _layer2_combine(
      h1, acc2[0], acc2[1], eagg[0], eagg[1], d0, d1,
      W_s2, W_n2, r1(b_s2),
      W_p1[128:256], W_p1[256:384], W_p1[512:640], W_p1[640:768], 1280)

  g = _sc_gather2(ph, pt, h_id, t_id)

  logits2d, y2d = _edge_score(
      batch_q_embds, edge_attr, g, gn,
      W_p1[0:128], W_p1[384:512], r1(b_p1), W_p2, b_p2.reshape(1, 1), 2560)

  ypad = jnp.concatenate(
      [y2d[:, 0], jnp.full((_EPAD - _E,), -jnp.inf, jnp.float32)]
  ).reshape(_EPAD // 128, 128)
  mask2d = _topk_mask(ypad)

  logits = logits2d[:, 0]
  mask = mask2d.reshape(-1)[:_E]
  return (logits, mask)
